# baseline (speedup 1.0000x reference)
import functools

import jax
import jax.numpy as jnp
from jax import lax
from jax.experimental import pallas as pl
from jax.experimental.pallas import tpu as pltpu
from jax.experimental.pallas import tpu_sc as plsc

_BATCH = 16384
_HIST = 50
_EMBED = 32
_NC = 2
_NS = 16
_NW = _NC * _NS
_RPW = _BATCH // _NW
_XB = 128
_NBLK = _RPW // _XB
_GR = 4
_NG = _XB // _GR
_GI = _GR * _HIST
_NB = 2
_K = 1


def _gather_body(x_hbm, w_hbm, out_hbm, idx_v, flat0, flat1, wide_v,
                 nrw_v, *sems):
    flats = (flat0, flat1)
    gsem = sems[:_NB]
    ssem = sems[_NB:]
    wid = lax.axis_index("s") * _NC + lax.axis_index("c")
    row0 = wid * _RPW

    for blk in range(_NBLK):
        base = row0 + blk * _XB

        def sanitize(g, b):
            for j in range(_GR):
                for c0 in (0, 16, 32, 34):
                    v = idx_v[g * _GR + j, pl.ds(c0, 16)]
                    flats[b][pl.ds(j * _HIST + c0, 16)] = v

        def gather(g, b):
            del g
            pltpu.async_copy(w_hbm.at[flats[b]], wide_v.at[b], gsem[b])

        def wait_gather(b):
            pltpu.make_async_copy(w_hbm.at[flats[b]], wide_v.at[b],
                                  gsem[b]).wait()

        def repack(b):
            def rp(i, c):
                for j in range(_GR):
                    for hh in range(10):
                        r = j * _HIST + i * 10 + hh
                        for c0 in (0, 16):
                            nrw_v[b, r, pl.ds(c0, 16)] = (
                                wide_v[b, r, pl.ds(c0, 16)])
                return c
            lax.fori_loop(0, _HIST // 10, rp, 0)

        def store(g, b):
            for j in range(_GR):
                pltpu.async_copy(nrw_v.at[b, pl.ds(j * _HIST, _HIST)],
                                 out_hbm.at[base + g * _GR + j], ssem[b])

        def wait_store(b):
            for j in range(_GR):
                pltpu.make_async_copy(nrw_v.at[b, pl.ds(0, _HIST)],
                                      out_hbm.at[0], ssem[b]).wait()

        def body(g, b, bk, with_ssem_wait):
            if with_ssem_wait:
                wait_store(bk)
            sanitize(g + _K, bk)
            gather(g + _K, bk)
            wait_gather(b)
            repack(b)
            store(g, b)

        pltpu.sync_copy(x_hbm.at[pl.ds(base, _XB)], idx_v)
        for g in range(_K):
            sanitize(g, g % _NB)
            gather(g, g % _NB)
        for g in range(_NB - _K):
            body(g, g % _NB, (g + _K) % _NB, False)
        for g in range(_NB - _K, _NB):
            body(g, g % _NB, (g + _K) % _NB, True)

        def outer(i, carry):
            for b in range(_NB):
                g = i * _NB + b
                body(g, b, (b + _K) % _NB, True)
            return carry

        lax.fori_loop(1, (_NG - _K) // _NB, outer, 0)

        for g in range(_NG - _NB, _NG - _K):
            body(g, g % _NB, (g + _K) % _NB, True)
        for g in range(_NG - _K, _NG):
            wait_gather(g % _NB)
            repack(g % _NB)
            store(g, g % _NB)
        for g in range(_NG - _NB, _NG):
            wait_store(g % _NB)


@jax.jit
def _embed(x, w_wide):
    k = functools.partial(
        pl.kernel,
        mesh=plsc.VectorSubcoreMesh(core_axis_name="c", subcore_axis_name="s"),
        out_type=jax.ShapeDtypeStruct((_BATCH, _HIST, _EMBED), jnp.float32),
        scratch_types=[
            pltpu.VMEM((_XB, _HIST), jnp.int32),
            pltpu.VMEM((_GI,), jnp.int32),
            pltpu.VMEM((_GI,), jnp.int32),
            pltpu.VMEM((_NB, _GI, 128), jnp.float32),
            pltpu.VMEM((_NB, _GI, _EMBED), jnp.float32),
        ] + [pltpu.SemaphoreType.DMA] * (2 * _NB),
    )(_gather_body)
    return k(x, w_wide)


def kernel(x, W):
    w_wide = jnp.pad(W, ((0, 0), (0, 128 - _EMBED)))
    return _embed(x, w_wide)


# speedup vs baseline: 1.1553x; 1.1553x over previous
import functools

import jax
import jax.numpy as jnp
from jax import lax
from jax.experimental import pallas as pl
from jax.experimental.pallas import tpu as pltpu
from jax.experimental.pallas import tpu_sc as plsc

_BATCH = 16384
_HIST = 50
_EMBED = 32
_NC = 2
_NS = 16
_NW = _NC * _NS
_RPW = _BATCH // _NW
_XB = 128
_NBLK = _RPW // _XB
_GR = 4
_NG = _XB // _GR
_GI = _GR * _HIST
_NB = 2
_K = 1


def _gather_body(x_hbm, w_hbm, out_hbm, idx_v, flat0, flat1, wide_v,
                 *sems):
    flats = (flat0, flat1)
    gsem = sems[:_NB]
    ssem = sems[_NB:]
    wid = lax.axis_index("s") * _NC + lax.axis_index("c")
    row0 = wid * _RPW

    for blk in range(_NBLK):
        base = row0 + blk * _XB

        def sanitize(g, b):
            for j in range(_GR):
                for c0 in (0, 16, 32, 34):
                    v = idx_v[g * _GR + j, pl.ds(c0, 16)]
                    flats[b][pl.ds(j * _HIST + c0, 16)] = v

        def gather(g, b):
            del g
            pltpu.async_copy(w_hbm.at[flats[b]], wide_v.at[b], gsem[b])

        def wait_gather(b):
            pltpu.make_async_copy(w_hbm.at[flats[b]], wide_v.at[b],
                                  gsem[b]).wait()

        def store(g, b):
            for j in range(_GR):
                pltpu.async_copy(wide_v.at[b, pl.ds(j * _HIST, _HIST)],
                                 out_hbm.at[base + g * _GR + j], ssem[b])

        def wait_store(b):
            for j in range(_GR):
                pltpu.make_async_copy(wide_v.at[b, pl.ds(0, _HIST)],
                                      out_hbm.at[0], ssem[b]).wait()

        def body(g, b, bk, with_ssem_wait):
            if with_ssem_wait:
                wait_store(bk)
            sanitize(g + _K, bk)
            gather(g + _K, bk)
            wait_gather(b)
            store(g, b)

        pltpu.sync_copy(x_hbm.at[pl.ds(base, _XB)], idx_v)
        for g in range(_K):
            sanitize(g, g % _NB)
            gather(g, g % _NB)
        for g in range(_NB - _K):
            body(g, g % _NB, (g + _K) % _NB, False)
        for g in range(_NB - _K, _NB):
            body(g, g % _NB, (g + _K) % _NB, True)

        def outer(i, carry):
            for b in range(_NB):
                g = i * _NB + b
                body(g, b, (b + _K) % _NB, True)
            return carry

        lax.fori_loop(1, (_NG - _K) // _NB, outer, 0)

        for g in range(_NG - _NB, _NG - _K):
            body(g, g % _NB, (g + _K) % _NB, True)
        for g in range(_NG - _K, _NG):
            wait_gather(g % _NB)
            store(g, g % _NB)
        for g in range(_NG - _NB, _NG):
            wait_store(g % _NB)


@jax.jit
def _embed(x, w_wide):
    k = functools.partial(
        pl.kernel,
        mesh=plsc.VectorSubcoreMesh(core_axis_name="c", subcore_axis_name="s"),
        out_type=jax.ShapeDtypeStruct((_BATCH, _HIST, 128), jnp.float32),
        scratch_types=[
            pltpu.VMEM((_XB, _HIST), jnp.int32),
            pltpu.VMEM((_GI,), jnp.int32),
            pltpu.VMEM((_GI,), jnp.int32),
            pltpu.VMEM((_NB, _GI, 128), jnp.float32),
        ] + [pltpu.SemaphoreType.DMA] * (2 * _NB),
    )(_gather_body)
    return k(x, w_wide)


def kernel(x, W):
    w_wide = jnp.pad(W, ((0, 0), (0, 128 - _EMBED)))
    return _embed(x, w_wide)[:, :, :_EMBED]


# speedup vs baseline: 1.1581x; 1.0025x over previous
import functools

import jax
import jax.numpy as jnp
from jax import lax
from jax.experimental import pallas as pl
from jax.experimental.pallas import tpu as pltpu
from jax.experimental.pallas import tpu_sc as plsc

_BATCH = 16384
_HIST = 50
_EMBED = 32
_NC = 2
_NS = 16
_NW = _NC * _NS
_RPW = _BATCH // _NW
_XB = 128
_NBLK = _RPW // _XB
_NB = 8
_K = 6


def _gather_body(x_hbm, w_hbm, out_hbm, idx_v, rows_v, *sems):
    gsem = sems[:_NB]
    ssem = sems[_NB:]
    wid = lax.axis_index("s") * _NC + lax.axis_index("c")
    row0 = wid * _RPW

    for blk in range(_NBLK):
        base = row0 + blk * _XB

        def gather(g, b):
            pltpu.async_copy(w_hbm.at[idx_v.at[g]], rows_v.at[b], gsem[b])

        def wait_gather(b):
            pltpu.make_async_copy(w_hbm.at[idx_v.at[0]], rows_v.at[b],
                                  gsem[b]).wait()

        def store(g, b):
            pltpu.async_copy(rows_v.at[b], out_hbm.at[base + g], ssem[b])

        def wait_store(b):
            pltpu.make_async_copy(rows_v.at[b], out_hbm.at[0],
                                  ssem[b]).wait()

        def body(g, b, bk, with_ssem_wait):
            if with_ssem_wait:
                wait_store(bk)
            gather(g + _K, bk)
            wait_gather(b)
            store(g, b)

        pltpu.sync_copy(x_hbm.at[pl.ds(base, _XB)], idx_v)
        for g in range(_K):
            gather(g, g)
        for g in range(_NB - _K):
            body(g, g % _NB, (g + _K) % _NB, False)
        for g in range(_NB - _K, _NB):
            body(g, g % _NB, (g + _K) % _NB, True)

        def outer(i, carry):
            for b in range(_NB):
                g = i * _NB + b
                body(g, b, (b + _K) % _NB, True)
            return carry

        lax.fori_loop(1, (_XB - _K) // _NB, outer, 0)

        for g in range(_XB - _NB, _XB - _K):
            body(g, g % _NB, (g + _K) % _NB, True)
        for g in range(_XB - _K, _XB):
            wait_gather(g % _NB)
            store(g, g % _NB)
        for g in range(_XB - _NB, _XB):
            wait_store(g % _NB)


@jax.jit
def _embed(x, w_wide):
    k = functools.partial(
        pl.kernel,
        mesh=plsc.VectorSubcoreMesh(core_axis_name="c", subcore_axis_name="s"),
        out_type=jax.ShapeDtypeStruct((_BATCH, _HIST, 128), jnp.float32),
        scratch_types=[
            pltpu.VMEM((_XB, _HIST), jnp.int32),
            pltpu.VMEM((_NB, _HIST, 128), jnp.float32),
        ] + [pltpu.SemaphoreType.DMA] * (2 * _NB),
    )(_gather_body)
    return k(x, w_wide)


def kernel(x, W):
    w_wide = jnp.pad(W, ((0, 0), (0, 128 - _EMBED)))
    return _embed(x, w_wide)[:, :, :_EMBED]


# speedup vs baseline: 1.1590x; 1.0008x over previous
import functools

import jax
import jax.numpy as jnp
from jax import lax
from jax.experimental import pallas as pl
from jax.experimental.pallas import tpu as pltpu
from jax.experimental.pallas import tpu_sc as plsc

_BATCH = 16384
_HIST = 50
_EMBED = 32
_NC = 2
_NS = 16
_NW = _NC * _NS
_RPW = _BATCH // _NW
_XB = 256
_NBLK = _RPW // _XB
_NB = 8
_K = 6


def _gather_body(x_hbm, w_hbm, out_hbm, idx_v, rows_v, *sems):
    gsem = sems[:_NB]
    ssem = sems[_NB:]
    wid = lax.axis_index("s") * _NC + lax.axis_index("c")
    row0 = wid * _RPW

    for blk in range(_NBLK):
        base = row0 + blk * _XB

        def gather(g, b):
            pltpu.async_copy(w_hbm.at[idx_v.at[g]], rows_v.at[b], gsem[b])

        def wait_gather(b):
            pltpu.make_async_copy(w_hbm.at[idx_v.at[0]], rows_v.at[b],
                                  gsem[b]).wait()

        def store(g, b):
            pltpu.async_copy(rows_v.at[b], out_hbm.at[base + g], ssem[b])

        def wait_store(b):
            pltpu.make_async_copy(rows_v.at[b], out_hbm.at[0],
                                  ssem[b]).wait()

        def body(g, b, bk, with_ssem_wait):
            if with_ssem_wait:
                wait_store(bk)
            gather(g + _K, bk)
            wait_gather(b)
            store(g, b)

        pltpu.sync_copy(x_hbm.at[pl.ds(base, _XB)], idx_v)
        for g in range(_K):
            gather(g, g)
        for g in range(_NB - _K):
            body(g, g % _NB, (g + _K) % _NB, False)
        for g in range(_NB - _K, _NB):
            body(g, g % _NB, (g + _K) % _NB, True)

        def outer(i, carry):
            for b in range(_NB):
                g = i * _NB + b
                body(g, b, (b + _K) % _NB, True)
            return carry

        lax.fori_loop(1, (_XB - _K) // _NB, outer, 0)

        for g in range(_XB - _NB, _XB - _K):
            body(g, g % _NB, (g + _K) % _NB, True)
        for g in range(_XB - _K, _XB):
            wait_gather(g % _NB)
            store(g, g % _NB)
        for g in range(_XB - _NB, _XB):
            wait_store(g % _NB)


@jax.jit
def _embed(x, w_wide):
    k = functools.partial(
        pl.kernel,
        mesh=plsc.VectorSubcoreMesh(core_axis_name="c", subcore_axis_name="s"),
        out_type=jax.ShapeDtypeStruct((_BATCH, _HIST, 128), jnp.float32),
        scratch_types=[
            pltpu.VMEM((_XB, _HIST), jnp.int32),
            pltpu.VMEM((_NB, _HIST, 128), jnp.float32),
        ] + [pltpu.SemaphoreType.DMA] * (2 * _NB),
    )(_gather_body)
    return k(x, w_wide)


def kernel(x, W):
    w_wide = jnp.pad(W, ((0, 0), (0, 128 - _EMBED)))
    return _embed(x, w_wide)[:, :, :_EMBED]
